# Initial kernel scaffold; baseline (speedup 1.0000x reference)
import jax
import jax.numpy as jnp
from jax.experimental import pallas as pl


def kernel(input_ids, table, mask_token):
    raise NotImplementedError("write your pallas kernel here")


# speedup vs baseline: 1.3578x; 1.3578x over previous
import functools
import math

import jax
import jax.numpy as jnp
import numpy as np
from jax import lax
from jax.experimental import pallas as pl
from jax.experimental.pallas import tpu as pltpu
from jax.experimental.pallas import tpu_sc as plsc

D_MODEL = 768
MASK_TOKEN_ID = 4

NUM_CORES = 2
NUM_SUBCORES = 16
NUM_WORKERS = NUM_CORES * NUM_SUBCORES
CHUNK = 64


def _pe_table(seq_len: int, d_model: int) -> np.ndarray:
    pos = np.arange(seq_len, dtype=np.float32)[:, None]
    div = np.exp(
        np.arange(0, d_model, 2, dtype=np.float32) * (-(math.log(10000.0) / d_model))
    )
    pe = np.zeros((seq_len, d_model), dtype=np.float32)
    pe[:, 0::2] = np.sin(pos * div)
    pe[:, 1::2] = np.cos(pos * div)
    return pe


@functools.cache
def _build_sc_gather(n_rows: int, seq_len: int, d: int, ext_row: int):
    rows_per_w = n_rows // NUM_WORKERS
    n_chunks = rows_per_w // CHUNK
    mesh = plsc.VectorSubcoreMesh(core_axis_name="c", subcore_axis_name="s")

    @functools.partial(
        pl.kernel,
        out_type=jax.ShapeDtypeStruct((n_rows, d), jnp.float32),
        mesh=mesh,
        scratch_types=[
            pltpu.VMEM((rows_per_w,), jnp.int32),
            pltpu.VMEM((CHUNK, d), jnp.float32),
            pltpu.VMEM((CHUNK, d), jnp.float32),
            pltpu.SemaphoreType.DMA,
        ],
    )
    def sc_gather(table_hbm, ids_hbm, pe_hbm, out_hbm, ids_v, buf_v, pe_v, gsem):
        wid = lax.axis_index("s") * NUM_CORES + lax.axis_index("c")
        base = wid * rows_per_w
        pos0 = lax.rem(base, seq_len)
        pltpu.sync_copy(ids_hbm.at[pl.ds(base, rows_per_w)], ids_v)
        for i in range(rows_per_w // 16):
            v = ids_v[pl.ds(i * 16, 16)]
            ids_v[pl.ds(i * 16, 16)] = jnp.where(
                v == MASK_TOKEN_ID, jnp.full_like(v, ext_row), v
            )
        for c in range(n_chunks):
            pltpu.sync_copy(pe_hbm.at[pl.ds(pos0 + c * CHUNK, CHUNK)], pe_v)
            pltpu.async_copy(
                table_hbm.at[ids_v.at[pl.ds(c * CHUNK, CHUNK)]], buf_v, gsem
            ).wait()

            def row_add(r, carry):
                br = buf_v.at[r]
                pr = pe_v.at[r]
                for g in range(d // 16):
                    plsc.addupdate(br.at[pl.ds(g * 16, 16)], pr[pl.ds(g * 16, 16)])
                return carry

            lax.fori_loop(0, CHUNK, row_add, 0)
            pltpu.sync_copy(buf_v, out_hbm.at[pl.ds(base + c * CHUNK, CHUNK)])

    return sc_gather


def kernel(input_ids, table, mask_token):
    b, l = input_ids.shape
    v, d = table.shape
    ext = jnp.concatenate([table, mask_token[None, :]], axis=0)
    ids = input_ids.reshape(-1).astype(jnp.int32)
    pe = jnp.asarray(_pe_table(l, d))
    out = _build_sc_gather(b * l, l, d, v)(ext, ids, pe)
    return out.reshape(b, l, d)


# speedup vs baseline: 1.6643x; 1.2258x over previous
import functools
import math

import jax
import jax.numpy as jnp
import numpy as np
from jax import lax
from jax.experimental import pallas as pl
from jax.experimental.pallas import tpu as pltpu
from jax.experimental.pallas import tpu_sc as plsc

D_MODEL = 768
MASK_TOKEN_ID = 4

NUM_CORES = 2
NUM_SUBCORES = 16
NUM_WORKERS = NUM_CORES * NUM_SUBCORES
CHUNK = 32


def _pe_table(seq_len: int, d_model: int) -> np.ndarray:
    pos = np.arange(seq_len, dtype=np.float32)[:, None]
    div = np.exp(
        np.arange(0, d_model, 2, dtype=np.float32) * (-(math.log(10000.0) / d_model))
    )
    pe = np.zeros((seq_len, d_model), dtype=np.float32)
    pe[:, 0::2] = np.sin(pos * div)
    pe[:, 1::2] = np.cos(pos * div)
    return pe


@functools.cache
def _build_sc_gather(n_rows: int, seq_len: int, d: int, ext_row: int):
    rows_per_w = n_rows // NUM_WORKERS
    n_chunks = rows_per_w // CHUNK
    mesh = plsc.VectorSubcoreMesh(core_axis_name="c", subcore_axis_name="s")

    @functools.partial(
        pl.kernel,
        out_type=jax.ShapeDtypeStruct((n_rows, d), jnp.float32),
        mesh=mesh,
        scratch_types=[
            pltpu.VMEM((rows_per_w,), jnp.int32),
            pltpu.VMEM((2, CHUNK, d), jnp.float32),
            pltpu.VMEM((3, CHUNK, d), jnp.float32),
        ]
        + [pltpu.SemaphoreType.DMA] * 8,
    )
    def sc_gather(table_hbm, ids_hbm, pe_hbm, out_hbm, ids_v, gbuf, pbuf, *sems):
        gsems = sems[0:2]
        psems = sems[2:5]
        osems = sems[5:8]
        wid = lax.axis_index("s") * NUM_CORES + lax.axis_index("c")
        base = wid * rows_per_w
        pos0 = lax.rem(base, seq_len)

        pltpu.sync_copy(ids_hbm.at[pl.ds(base, rows_per_w)], ids_v)
        for i in range(rows_per_w // 16):
            v = ids_v[pl.ds(i * 16, 16)]
            ids_v[pl.ds(i * 16, 16)] = jnp.where(
                v == MASK_TOKEN_ID, jnp.full_like(v, ext_row), v
            )

        def start_gather(c):
            return pltpu.async_copy(
                table_hbm.at[ids_v.at[pl.ds(c * CHUNK, CHUNK)]],
                gbuf.at[c % 2],
                gsems[c % 2],
            )

        def start_pe(c):
            return pltpu.async_copy(
                pe_hbm.at[pl.ds(pos0 + c * CHUNK, CHUNK)],
                pbuf.at[c % 3],
                psems[c % 3],
            )

        def start_out(c):
            return pltpu.async_copy(
                pbuf.at[c % 3],
                out_hbm.at[pl.ds(base + c * CHUNK, CHUNK)],
                osems[c % 3],
            )

        g_desc = [None] * n_chunks
        p_desc = [None] * n_chunks
        o_desc = [None] * n_chunks
        for c in range(min(2, n_chunks)):
            p_desc[c] = start_pe(c)
            g_desc[c] = start_gather(c)

        for c in range(n_chunks):
            p_desc[c].wait()
            g_desc[c].wait()
            src = gbuf.at[c % 2]
            dst = pbuf.at[c % 3]

            def row_add(r, carry, src=src, dst=dst):
                sr = src.at[r]
                dr = dst.at[r]
                for g in range(d // 16):
                    plsc.addupdate(dr.at[pl.ds(g * 16, 16)], sr[pl.ds(g * 16, 16)])
                return carry

            lax.fori_loop(0, CHUNK, row_add, 0)
            if c + 2 < n_chunks:
                g_desc[c + 2] = start_gather(c + 2)
            o_desc[c] = start_out(c)
            if c + 2 < n_chunks:
                if c - 1 >= 0:
                    o_desc[c - 1].wait()
                p_desc[c + 2] = start_pe(c + 2)
        for c in range(max(0, n_chunks - 2), n_chunks):
            o_desc[c].wait()

    return sc_gather


def kernel(input_ids, table, mask_token):
    b, l = input_ids.shape
    v, d = table.shape
    ext = jnp.concatenate([table, mask_token[None, :]], axis=0)
    ids = input_ids.reshape(-1).astype(jnp.int32)
    pe = jnp.asarray(_pe_table(l, d))
    out = _build_sc_gather(b * l, l, d, v)(ext, ids, pe)
    return out.reshape(b, l, d)
